# baseline (speedup 1.0000x reference)
import functools

import jax
import jax.numpy as jnp
from jax import lax
from jax.experimental import pallas as pl
from jax.experimental.pallas import tpu as pltpu
from jax.experimental.pallas import tpu_sc as plsc

_USER = 20000
_ITEM = 30000
_N = 50000
_D = 64
_E = 800000
_LAYERS = 3

_NC = 2
_NS = 16

_HALF = _N // _NC
_SENT = _HALF
_TSLICE = 1600
_ACC_ROWS = 25008

_OUTER = 640
_SUB = 80
_NSUB = _OUTER // _SUB
_TOTAL_OUTER = (_E + _OUTER - 1) // _OUTER
_E_PAD = _TOTAL_OUTER * _OUTER
_TOTAL_SUB = _E_PAD // _SUB

_mesh = plsc.VectorSubcoreMesh(
    core_axis_name="c", subcore_axis_name="s", num_cores=_NC, num_subcores=_NS
)


@functools.partial(
    pl.kernel,
    out_type=jax.ShapeDtypeStruct((_N, _D), jnp.float32),
    mesh=_mesh,
    scratch_types=[
        pltpu.VMEM((_NSUB, _SUB), jnp.int32),
        pltpu.VMEM((_OUTER,), jnp.float32),
        pltpu.VMEM((_OUTER,), jnp.int32),
        pltpu.VMEM((_NSUB, _SUB), jnp.int32),
        pltpu.VMEM((_OUTER,), jnp.float32),
        pltpu.VMEM((_OUTER,), jnp.int32),
        pltpu.VMEM((_NSUB, _SUB), jnp.int32),
        pltpu.VMEM((_SUB, _D), jnp.float32),
        pltpu.VMEM((_SUB, _D), jnp.float32),
        pltpu.VMEM((_SUB, _D), jnp.float32),
        pltpu.VMEM((_SUB, _D), jnp.float32),
        pltpu.VMEM((_SUB, _D), jnp.float32),
        pltpu.VMEM((16,), jnp.int32),
        pltpu.VMEM_SHARED((_ACC_ROWS, _D), jnp.float32),
        pltpu.SemaphoreType.DMA,
        pltpu.SemaphoreType.DMA,
        pltpu.SemaphoreType.DMA,
        pltpu.SemaphoreType.DMA,
        pltpu.SemaphoreType.DMA,
        pltpu.SemaphoreType.DMA,
        pltpu.SemaphoreType.DMA,
    ],
    compiler_params=pltpu.CompilerParams(use_tc_tiling_on_sc=False),
)
def _layer(ego_hbm, cols_hbm, vals_hbm, rows_hbm, bnd_hbm, out_hbm,
           colsv0, valsv0, rowsv0, colsv1, valsv1, rowsv1, lidx,
           gbuf0, gbuf1, gbuf2, sbuf0, sbuf1, bndv, acc,
           stg0, stg1, semg0, semg1, semg2, sems0, sems1):
    sc = lax.axis_index("c")
    sid = lax.axis_index("s")

    def _zrow(r, carry):
        for c in range(_D // 16):
            gbuf0[r, pl.ds(c * 16, 16)] = jnp.zeros((16,), jnp.float32)
        return carry

    lax.fori_loop(0, _SUB, _zrow, 0)
    zstart = jnp.minimum(sid * _TSLICE, _ACC_ROWS - _TSLICE)
    for j in range(_TSLICE // _SUB):
        pltpu.sync_copy(gbuf0, acc.at[pl.ds(zstart + j * _SUB, _SUB)])
    plsc.subcore_barrier()

    pltpu.sync_copy(bnd_hbm, bndv)
    bedge = bndv[pl.ds(0, 16)][0]
    lo = jnp.where(sc == 0, 0, bedge // _OUTER)
    hi = jnp.where(sc == 0, (bedge + _OUTER - 1) // _OUTER, _TOTAL_OUTER)
    base_row = sc * _HALF
    n_iter = jnp.maximum(0, (hi - lo - sid + _NS - 1) // _NS)

    def _stage_refs(oc):
        return (
            (cols_hbm.at[pl.ds(oc * _NSUB, _NSUB)],),
            (vals_hbm.at[pl.ds(oc * _OUTER, _OUTER)],),
            (rows_hbm.at[pl.ds(oc * _OUTER, _OUTER)],),
        )

    def _issue_stage(oc, cv, vv, rv, sem):
        (c_src,), (v_src,), (r_src,) = _stage_refs(oc)
        pltpu.async_copy(c_src, cv, sem)
        pltpu.async_copy(v_src, vv, sem)
        pltpu.async_copy(r_src, rv, sem)

    def _wait_stage(oc, cv, vv, rv, sem):
        (c_src,), (v_src,), (r_src,) = _stage_refs(oc)
        pltpu.make_async_copy(c_src, cv, sem).wait()
        pltpu.make_async_copy(v_src, vv, sem).wait()
        pltpu.make_async_copy(r_src, rv, sem).wait()

    oc_first = jnp.minimum(lo + sid, _TOTAL_OUTER - 1)

    @pl.when(n_iter > 0)
    def _prime():
        _issue_stage(oc_first, colsv0, valsv0, rowsv0, stg0)

    stag = (
        (colsv0, valsv0, rowsv0, stg0),
        (colsv1, valsv1, rowsv1, stg1),
    )

    def _pair(m, carry):
        for half in range(2):
            cv, vv, rv, st_sem = stag[half]
            ncv, nvv, nrv, nst_sem = stag[1 - half]
            k = m * 2 + half

            @pl.when(k < n_iter)
            def _chunk(k=k, cv=cv, vv=vv, rv=rv, st_sem=st_sem,
                       ncv=ncv, nvv=nvv, nrv=nrv, nst_sem=nst_sem):
                oc = lo + sid + k * _NS
                _wait_stage(oc, cv, vv, rv, st_sem)

                ocn = jnp.minimum(
                    lo + sid + (k + 1) * _NS, _TOTAL_OUTER - 1
                )

                @pl.when(k + 1 < n_iter)
                def _prefetch():
                    _issue_stage(ocn, ncv, nvv, nrv, nst_sem)

                for j in range(_NSUB):
                    for q in range(_SUB // 16):
                        r16 = rv[pl.ds(j * _SUB + q * 16, 16)]
                        loc = r16 - base_row
                        ok = (loc >= 0) & (loc < _HALF)
                        lidx[j, pl.ds(q * 16, 16)] = jnp.where(
                            ok, loc, _SENT
                        )

                gb = (gbuf0, gbuf1, gbuf2)
                sb = (sbuf0, sbuf1)
                gsem = (semg0, semg1, semg2)
                ssem = (sems0, sems1)
                gd = [
                    pltpu.async_copy(
                        ego_hbm.at[cv.at[j]], gb[j], gsem[j]
                    )
                    for j in range(2)
                ]
                sd = [None, None]
                for j in range(_NSUB):
                    b3 = j % 2
                    b2 = j % 2
                    gd[b3].wait()
                    if sd[b2] is not None:
                        sd[b2].wait()

                    def _scale(g, c2, _j=j, _b3=b3, _b2=b2):
                        v16 = vv[pl.ds(_j * _SUB + g * 16, 16)]
                        for el in range(16):
                            v = v16[el]
                            e = g * 16 + el
                            for c in range(_D // 16):
                                sb[_b2][e, pl.ds(c * 16, 16)] = (
                                    gb[_b3][e, pl.ds(c * 16, 16)] * v
                                )
                        return c2

                    lax.fori_loop(0, _SUB // 16, _scale, 0)
                    if j + 2 < _NSUB:
                        gd[b3] = pltpu.async_copy(
                            ego_hbm.at[cv.at[j + 2]], gb[b3], gsem[b3]
                        )
                    sd[b2] = pltpu.async_copy(
                        sb[b2], acc.at[lidx.at[j]], ssem[b2], add=True
                    )
                sd[0].wait()
                sd[1].wait()

        return carry

    lax.fori_loop(0, (n_iter + 1) // 2, _pair, 0)
    plsc.subcore_barrier()

    start = jnp.minimum(sid * _TSLICE, _HALF - _TSLICE)
    pltpu.sync_copy(
        acc.at[pl.ds(start, _TSLICE)],
        out_hbm.at[pl.ds(base_row + start, _TSLICE)],
    )


def _comb_body(a_ref, b_ref, c_ref, o_ref):
    o_ref[...] = (a_ref[...] + b_ref[...] + c_ref[...]) * (1.0 / 3.0)


def _combine(e1, e2, e3):
    blk = 1000
    grid = _N // blk
    spec = pl.BlockSpec((blk, _D), lambda i: (i, 0))
    return pl.pallas_call(
        _comb_body,
        grid=(grid,),
        in_specs=[spec, spec, spec],
        out_specs=spec,
        out_shape=jax.ShapeDtypeStruct((_N, _D), jnp.float32),
    )(e1, e2, e3)


def kernel(user_emb, item_emb, adj_vals, adj_rows, adj_cols):
    ego = jnp.concatenate([user_emb, item_emb], axis=0)
    pad = _E_PAD - _E
    if pad:
        cols_f = jnp.concatenate(
            [adj_cols.astype(jnp.int32), jnp.zeros((pad,), jnp.int32)]
        )
        vals_p = jnp.concatenate([adj_vals, jnp.zeros((pad,), jnp.float32)])
        rows_p = jnp.concatenate(
            [adj_rows.astype(jnp.int32), jnp.full((pad,), _N - 1, jnp.int32)]
        )
    else:
        cols_f = adj_cols.astype(jnp.int32)
        vals_p = adj_vals
        rows_p = adj_rows.astype(jnp.int32)
    cols_p = cols_f.reshape(_TOTAL_SUB, _SUB)
    bedge = jnp.searchsorted(rows_p, _HALF).astype(jnp.int32)
    bnd = jnp.zeros((16,), jnp.int32).at[0].set(bedge)

    outs = []
    cur = ego
    for _ in range(_LAYERS):
        cur = _layer(cur, cols_p, vals_p, rows_p, bnd)
        outs.append(cur)
    all_e = _combine(*outs)
    return all_e[:_USER], all_e[_USER:]


# speedup vs baseline: 1.1541x; 1.1541x over previous
import functools

import jax
import jax.numpy as jnp
from jax import lax
from jax.experimental import pallas as pl
from jax.experimental.pallas import tpu as pltpu
from jax.experimental.pallas import tpu_sc as plsc

_USER = 20000
_ITEM = 30000
_N = 50000
_D = 64
_E = 800000
_LAYERS = 3

_NC = 2
_NS = 16

_HALF = _N // _NC
_SENT = _HALF
_TSLICE = 1632
_ACC_ROWS = 25008

_OUTER = 768
_SUB = 96
_NSUB = _OUTER // _SUB
_TOTAL_OUTER = (_E + _OUTER - 1) // _OUTER
_E_PAD = _TOTAL_OUTER * _OUTER
_TOTAL_SUB = _E_PAD // _SUB

_mesh = plsc.VectorSubcoreMesh(
    core_axis_name="c", subcore_axis_name="s", num_cores=_NC, num_subcores=_NS
)


@functools.partial(
    pl.kernel,
    out_type=jax.ShapeDtypeStruct((_N, _D), jnp.float32),
    mesh=_mesh,
    scratch_types=[
        pltpu.VMEM((_NSUB, _SUB), jnp.int32),
        pltpu.VMEM((_OUTER,), jnp.float32),
        pltpu.VMEM((_OUTER,), jnp.int32),
        pltpu.VMEM((_NSUB, _SUB), jnp.int32),
        pltpu.VMEM((_OUTER,), jnp.float32),
        pltpu.VMEM((_OUTER,), jnp.int32),
        pltpu.VMEM((_NSUB, _SUB), jnp.int32),
        pltpu.VMEM((_SUB, _D), jnp.float32),
        pltpu.VMEM((_SUB, _D), jnp.float32),
        pltpu.VMEM((_SUB, _D), jnp.float32),
        pltpu.VMEM((_SUB, _D), jnp.float32),
        pltpu.VMEM((16,), jnp.int32),
        pltpu.VMEM_SHARED((_ACC_ROWS, _D), jnp.float32),
        pltpu.SemaphoreType.DMA,
        pltpu.SemaphoreType.DMA,
        pltpu.SemaphoreType.DMA,
        pltpu.SemaphoreType.DMA,
        pltpu.SemaphoreType.DMA,
        pltpu.SemaphoreType.DMA,
    ],
    compiler_params=pltpu.CompilerParams(use_tc_tiling_on_sc=False),
)
def _layer(ego_hbm, cols_hbm, vals_hbm, rows_hbm, bnd_hbm, out_hbm,
           colsv0, valsv0, rowsv0, colsv1, valsv1, rowsv1, lidx,
           gbuf0, gbuf1, sbuf0, sbuf1, bndv, acc,
           stg0, stg1, semg0, semg1, sems0, sems1):
    sc = lax.axis_index("c")
    sid = lax.axis_index("s")

    def _zrow(r, carry):
        for c in range(_D // 16):
            gbuf0[r, pl.ds(c * 16, 16)] = jnp.zeros((16,), jnp.float32)
        return carry

    lax.fori_loop(0, _SUB, _zrow, 0)
    zstart = jnp.minimum(sid * _TSLICE, _ACC_ROWS - _TSLICE)
    for j in range(_TSLICE // _SUB):
        pltpu.sync_copy(gbuf0, acc.at[pl.ds(zstart + j * _SUB, _SUB)])
    plsc.subcore_barrier()

    pltpu.sync_copy(bnd_hbm, bndv)
    bedge = bndv[pl.ds(0, 16)][0]
    lo = jnp.where(sc == 0, 0, bedge // _OUTER)
    hi = jnp.where(sc == 0, (bedge + _OUTER - 1) // _OUTER, _TOTAL_OUTER)
    base_row = sc * _HALF
    n_iter = jnp.maximum(0, (hi - lo - sid + _NS - 1) // _NS)

    def _stage_refs(oc):
        return (
            (cols_hbm.at[pl.ds(oc * _NSUB, _NSUB)],),
            (vals_hbm.at[pl.ds(oc * _OUTER, _OUTER)],),
            (rows_hbm.at[pl.ds(oc * _OUTER, _OUTER)],),
        )

    def _issue_stage(oc, cv, vv, rv, sem):
        (c_src,), (v_src,), (r_src,) = _stage_refs(oc)
        pltpu.async_copy(c_src, cv, sem)
        pltpu.async_copy(v_src, vv, sem)
        pltpu.async_copy(r_src, rv, sem)

    def _wait_stage(oc, cv, vv, rv, sem):
        (c_src,), (v_src,), (r_src,) = _stage_refs(oc)
        pltpu.make_async_copy(c_src, cv, sem).wait()
        pltpu.make_async_copy(v_src, vv, sem).wait()
        pltpu.make_async_copy(r_src, rv, sem).wait()

    oc_first = jnp.minimum(lo + sid, _TOTAL_OUTER - 1)

    @pl.when(n_iter > 0)
    def _prime():
        _issue_stage(oc_first, colsv0, valsv0, rowsv0, stg0)
        _wait_stage(oc_first, colsv0, valsv0, rowsv0, stg0)
        pltpu.async_copy(ego_hbm.at[colsv0.at[0]], gbuf0, semg0)
        pltpu.async_copy(ego_hbm.at[colsv0.at[1]], gbuf1, semg1)

    stag = (
        (colsv0, valsv0, rowsv0, stg0),
        (colsv1, valsv1, rowsv1, stg1),
    )

    def _pair(m, carry):
        for half in range(2):
            cv, vv, rv, st_sem = stag[half]
            ncv, nvv, nrv, nst_sem = stag[1 - half]
            k = m * 2 + half

            @pl.when(k < n_iter)
            def _chunk(k=k, cv=cv, vv=vv, rv=rv, st_sem=st_sem,
                       ncv=ncv, nvv=nvv, nrv=nrv, nst_sem=nst_sem):
                oc = lo + sid + k * _NS
                ocn = jnp.minimum(
                    lo + sid + (k + 1) * _NS, _TOTAL_OUTER - 1
                )

                @pl.when(k + 1 < n_iter)
                def _prefetch():
                    _issue_stage(ocn, ncv, nvv, nrv, nst_sem)

                for j in range(_NSUB):
                    for q in range(_SUB // 16):
                        r16 = rv[pl.ds(j * _SUB + q * 16, 16)]
                        loc = r16 - base_row
                        ok = (loc >= 0) & (loc < _HALF)
                        lidx[j, pl.ds(q * 16, 16)] = jnp.where(
                            ok, loc, _SENT
                        )

                gb = (gbuf0, gbuf1)
                sb = (sbuf0, sbuf1)
                gsem = (semg0, semg1)
                ssem = (sems0, sems1)
                sd = [None, None]
                for j in range(_NSUB):
                    b = j % 2
                    pltpu.make_async_copy(
                        ego_hbm.at[cv.at[j]], gb[b], gsem[b]
                    ).wait()
                    if sd[b] is not None:
                        sd[b].wait()

                    def _scale(g, c2, _j=j, _b=b):
                        v16 = vv[pl.ds(_j * _SUB + g * 16, 16)]
                        for el in range(16):
                            v = v16[el]
                            e = g * 16 + el
                            for c in range(_D // 16):
                                sb[_b][e, pl.ds(c * 16, 16)] = (
                                    gb[_b][e, pl.ds(c * 16, 16)] * v
                                )
                        return c2

                    lax.fori_loop(0, _SUB // 16, _scale, 0)
                    if j + 2 < _NSUB:
                        pltpu.async_copy(
                            ego_hbm.at[cv.at[j + 2]], gb[b], gsem[b]
                        )
                    else:
                        @pl.when(k + 1 < n_iter)
                        def _tail(j=j, b=b):
                            if j == _NSUB - 2:
                                _wait_stage(ocn, ncv, nvv, nrv, nst_sem)
                            pltpu.async_copy(
                                ego_hbm.at[ncv.at[j - (_NSUB - 2)]],
                                gb[b], gsem[b],
                            )
                    sd[b] = pltpu.async_copy(
                        sb[b], acc.at[lidx.at[j]], ssem[b], add=True
                    )
                sd[0].wait()
                sd[1].wait()

        return carry

    lax.fori_loop(0, (n_iter + 1) // 2, _pair, 0)
    plsc.subcore_barrier()

    start = jnp.minimum(sid * _TSLICE, _HALF - _TSLICE)
    pltpu.sync_copy(
        acc.at[pl.ds(start, _TSLICE)],
        out_hbm.at[pl.ds(base_row + start, _TSLICE)],
    )


def _comb_body(a_ref, b_ref, c_ref, o_ref):
    o_ref[...] = (a_ref[...] + b_ref[...] + c_ref[...]) * (1.0 / 3.0)


def _combine(e1, e2, e3):
    blk = 1000
    grid = _N // blk
    spec = pl.BlockSpec((blk, _D), lambda i: (i, 0))
    return pl.pallas_call(
        _comb_body,
        grid=(grid,),
        in_specs=[spec, spec, spec],
        out_specs=spec,
        out_shape=jax.ShapeDtypeStruct((_N, _D), jnp.float32),
    )(e1, e2, e3)


def kernel(user_emb, item_emb, adj_vals, adj_rows, adj_cols):
    ego = jnp.concatenate([user_emb, item_emb], axis=0)
    pad = _E_PAD - _E
    cols_p = jnp.concatenate(
        [adj_cols.astype(jnp.int32), jnp.zeros((pad,), jnp.int32)]
    ).reshape(_TOTAL_SUB, _SUB)
    vals_p = jnp.concatenate([adj_vals, jnp.zeros((pad,), jnp.float32)])
    rows_p = jnp.concatenate(
        [adj_rows.astype(jnp.int32), jnp.full((pad,), _N - 1, jnp.int32)]
    )
    bedge = jnp.searchsorted(rows_p, _HALF).astype(jnp.int32)
    bnd = jnp.zeros((16,), jnp.int32).at[0].set(bedge)

    outs = []
    cur = ego
    for _ in range(_LAYERS):
        cur = _layer(cur, cols_p, vals_p, rows_p, bnd)
        outs.append(cur)
    all_e = _combine(*outs)
    return all_e[:_USER], all_e[_USER:]
